# baseline (speedup 1.0000x reference)
import jax
import jax.numpy as jnp
from jax.experimental import pallas as pl
from jax.experimental.pallas import tpu as pltpu


def _copy_body(w_ref, o_ref, sem):
    copy = pltpu.make_async_copy(w_ref, o_ref.at[0], sem)
    copy.start()
    copy.wait()


def kernel(x, emb_weight):
    seq_len = x.shape[1]
    out = pl.pallas_call(
        _copy_body,
        out_shape=jax.ShapeDtypeStruct(
            (1, seq_len, emb_weight.shape[1]), emb_weight.dtype
        ),
        in_specs=[pl.BlockSpec(memory_space=pl.ANY)],
        out_specs=pl.BlockSpec(memory_space=pl.ANY),
        scratch_shapes=[pltpu.SemaphoreType.DMA],
    )(emb_weight)
    return out
